# baseline (speedup 1.0000x reference)
import jax
import jax.numpy as jnp
from jax.experimental import pallas as pl

ALPHA = 16.0
K = 32
INT_MIN = -(2 ** 31)


def _key_map(v):
    return jnp.bitwise_xor(
        v, jnp.bitwise_and(v >> 31, jnp.int32(0x7FFFFFFF)))


def _msb_search_i32(data, n):
    def body(i, t):
        bit = jax.lax.shift_left(jnp.int32(1), jnp.int32(31) - i)
        tt = t + bit
        cnt = jnp.count_nonzero(data < tt, axis=1, keepdims=True)
        return jnp.where(cnt <= K, tt, t)

    t0 = jnp.full((n, 1), INT_MIN, jnp.int32)
    return jax.lax.fori_loop(0, 32, body, t0, unroll=16)


def _nca_kernel(x_ref, tcol_ref, trow_ref, loss_ref, mp_ref, mn_ref):
    n = x_ref.shape[0]
    x = x_ref[...]
    sim = jax.lax.dot_general(
        x, x, (((1,), (1,)), ((), ())),
        preferred_element_type=jnp.float32)
    tcol = tcol_ref[...]
    trow = trow_ref[...]
    same = tcol == trow
    lt_one = sim < 1.0
    pos_mask = same & lt_one
    masked = jnp.where(same, jnp.where(lt_one, sim, jnp.float32(2.0)), sim)
    skey = _key_map(jax.lax.bitcast_convert_type(masked, jnp.int32))

    tkey = _msb_search_i32(skey, n)
    thr = jax.lax.bitcast_convert_type(_key_map(tkey), jnp.float32)

    below = sim < thr
    colsum = jnp.sum(x, axis=0, keepdims=True)
    base = jax.lax.dot_general(
        x, colsum, (((1,), (1,)), ((), ())),
        preferred_element_type=jnp.float32) / jnp.float32(n)
    expt = jnp.exp(ALPHA * (base - sim))
    zero = jnp.float32(0.0)
    pos_logit = jnp.sum(jnp.where(pos_mask & below, expt, zero),
                        axis=1, keepdims=True)
    tot_logit = jnp.sum(jnp.where(below & (pos_mask | ~same), expt, zero),
                        axis=1, keepdims=True)
    neg_logit = tot_logit - pos_logit
    min_pos = jnp.min(jnp.where(pos_mask, sim, jnp.inf), axis=1, keepdims=True)
    pos_fb = jnp.exp(ALPHA * (base - min_pos))
    pos_logit = jnp.where(pos_logit == zero, pos_fb, pos_logit)
    loss_i = -jnp.log(pos_logit / (pos_logit + neg_logit))
    loss_ref[...] = jnp.sum(loss_i, keepdims=True).reshape(1, 1) / jnp.float32(n)

    sim_last = sim[n - 1:n, :]
    pos_last = pos_mask[n - 1:n, :]
    neg_last = jnp.logical_not(same[n - 1:n, :])
    ps = jnp.sum(jnp.where(pos_last, sim_last, zero), axis=1, keepdims=True)
    pc = jnp.sum(pos_last.astype(jnp.float32), axis=1, keepdims=True)
    ns = jnp.sum(jnp.where(neg_last, sim_last, zero), axis=1, keepdims=True)
    nc = jnp.sum(neg_last.astype(jnp.float32), axis=1, keepdims=True)
    mp_ref[...] = ps / jnp.maximum(pc, 1.0)
    mn_ref[...] = ns / jnp.maximum(nc, 1.0)


def kernel(inputs, targets):
    n = inputs.shape[0]
    tcol = targets.reshape(n, 1)
    trow = targets.reshape(1, n)
    out_shape = [jax.ShapeDtypeStruct((1, 1), jnp.float32)] * 3
    loss, mp, mn = pl.pallas_call(
        _nca_kernel,
        out_shape=out_shape,
    )(inputs, tcol, trow)
    return loss[0, 0], jnp.float32(0.0), mp[0, 0], mn[0, 0]


# speedup vs baseline: 1.0210x; 1.0210x over previous
import jax
import jax.numpy as jnp
from jax.experimental import pallas as pl

ALPHA = 16.0
K = 32
INT_MIN = -(2 ** 31)


def _f32_to_key(f):
    b = jax.lax.bitcast_convert_type(f, jnp.int32)
    m = jnp.int32(INT_MIN)
    return jnp.where(b >= 0, b, jnp.bitwise_xor(jnp.bitwise_not(b), m))


def _key_to_f32(k):
    m = jnp.int32(INT_MIN)
    b = jnp.where(k >= 0, k, jnp.bitwise_not(jnp.bitwise_xor(k, m)))
    return jax.lax.bitcast_convert_type(b, jnp.float32)


def _msb_search_i32(data, n):
    def body(i, t):
        bit = jax.lax.shift_left(jnp.int32(1), jnp.int32(31) - i)
        tt = t + bit
        cnt = jnp.sum((data < tt).astype(jnp.int32), axis=1, keepdims=True)
        return jnp.where(cnt <= K, tt, t)

    t0 = jnp.full((n, 1), INT_MIN, jnp.int32)
    return jax.lax.fori_loop(0, 32, body, t0, unroll=32)


def _nca_kernel(x_ref, tcol_ref, trow_ref, loss_ref, mp_ref, mn_ref):
    n = x_ref.shape[0]
    x = x_ref[...]
    sim = jax.lax.dot_general(
        x, x, (((1,), (1,)), ((), ())),
        preferred_element_type=jnp.float32)
    tcol = tcol_ref[...]
    trow = trow_ref[...]
    same = tcol == trow
    pos_mask = same & (sim < 1.0)
    neg_mask = jnp.logical_not(same)
    valid = pos_mask | neg_mask
    masked = jnp.where(valid, sim, jnp.float32(2.0))
    skey = _f32_to_key(masked)

    tkey = _msb_search_i32(skey, n)
    thr = _key_to_f32(tkey)

    below = sim < thr
    base = jnp.sum(sim, axis=1, keepdims=True) / jnp.float32(n)
    expt = jnp.exp(ALPHA * (base - sim))
    zero = jnp.float32(0.0)
    pos_logit = jnp.sum(jnp.where(pos_mask & below, expt, zero),
                        axis=1, keepdims=True)
    neg_logit = jnp.sum(jnp.where(neg_mask & below, expt, zero),
                        axis=1, keepdims=True)
    min_pos = jnp.min(jnp.where(pos_mask, sim, jnp.inf), axis=1, keepdims=True)
    pos_fb = jnp.exp(ALPHA * (base - min_pos))
    pos_logit = jnp.where(pos_logit == zero, pos_fb, pos_logit)
    loss_i = -jnp.log(pos_logit / (pos_logit + neg_logit))
    loss_ref[...] = jnp.sum(loss_i, keepdims=True).reshape(1, 1) / jnp.float32(n)

    sim_last = sim[n - 1:n, :]
    pos_last = pos_mask[n - 1:n, :]
    neg_last = neg_mask[n - 1:n, :]
    ps = jnp.sum(jnp.where(pos_last, sim_last, zero), axis=1, keepdims=True)
    pc = jnp.sum(pos_last.astype(jnp.float32), axis=1, keepdims=True)
    ns = jnp.sum(jnp.where(neg_last, sim_last, zero), axis=1, keepdims=True)
    nc = jnp.sum(neg_last.astype(jnp.float32), axis=1, keepdims=True)
    mp_ref[...] = ps / jnp.maximum(pc, 1.0)
    mn_ref[...] = ns / jnp.maximum(nc, 1.0)


def kernel(inputs, targets):
    n = inputs.shape[0]
    tcol = targets.reshape(n, 1)
    trow = targets.reshape(1, n)
    out_shape = [jax.ShapeDtypeStruct((1, 1), jnp.float32)] * 3
    loss, mp, mn = pl.pallas_call(
        _nca_kernel,
        out_shape=out_shape,
    )(inputs, tcol, trow)
    return loss[0, 0], jnp.float32(0.0), mp[0, 0], mn[0, 0]


# speedup vs baseline: 1.0451x; 1.0236x over previous
import jax
import jax.numpy as jnp
from jax.experimental import pallas as pl

ALPHA = 16.0
K = 32
INT_MIN = -(2 ** 31)


def _f32_to_key(f):
    b = jax.lax.bitcast_convert_type(f, jnp.int32)
    m = jnp.int32(INT_MIN)
    return jnp.where(b >= 0, b, jnp.bitwise_xor(jnp.bitwise_not(b), m))


def _key_to_f32(k):
    m = jnp.int32(INT_MIN)
    b = jnp.where(k >= 0, k, jnp.bitwise_not(jnp.bitwise_xor(k, m)))
    return jax.lax.bitcast_convert_type(b, jnp.float32)


KEY_LO = -0x3F810000


def _msb_search_i32(data, n):
    def body(i, t):
        bit = jax.lax.shift_left(jnp.int32(1), jnp.int32(30) - i)
        tt = t + bit
        cnt = jnp.sum((data < tt).astype(jnp.int32), axis=1, keepdims=True)
        return jnp.where(cnt <= K, tt, t)

    t0 = jnp.full((n, 1), KEY_LO, jnp.int32)
    return jax.lax.fori_loop(0, 31, body, t0, unroll=31)


def _nca_kernel(x_ref, tcol_ref, trow_ref, loss_ref, mp_ref, mn_ref):
    n = x_ref.shape[0]
    x = x_ref[...]
    sim = jax.lax.dot_general(
        x, x, (((1,), (1,)), ((), ())),
        preferred_element_type=jnp.float32)
    tcol = tcol_ref[...]
    trow = trow_ref[...]
    same = tcol == trow
    pos_mask = same & (sim < 1.0)
    neg_mask = jnp.logical_not(same)
    valid = pos_mask | neg_mask
    masked = jnp.where(valid, sim, jnp.float32(2.0))
    skey = _f32_to_key(masked)

    tkey = _msb_search_i32(skey, n)
    thr = _key_to_f32(tkey)

    below = sim < thr
    base = jnp.sum(sim, axis=1, keepdims=True) / jnp.float32(n)
    expt = jnp.exp(ALPHA * (base - sim))
    zero = jnp.float32(0.0)
    pos_logit = jnp.sum(jnp.where(pos_mask & below, expt, zero),
                        axis=1, keepdims=True)
    neg_logit = jnp.sum(jnp.where(neg_mask & below, expt, zero),
                        axis=1, keepdims=True)
    min_pos = jnp.min(jnp.where(pos_mask, sim, jnp.inf), axis=1, keepdims=True)
    pos_fb = jnp.exp(ALPHA * (base - min_pos))
    pos_logit = jnp.where(pos_logit == zero, pos_fb, pos_logit)
    loss_i = -jnp.log(pos_logit / (pos_logit + neg_logit))
    loss_ref[...] = jnp.sum(loss_i, keepdims=True).reshape(1, 1) / jnp.float32(n)

    sim_last = sim[n - 1:n, :]
    pos_last = pos_mask[n - 1:n, :]
    neg_last = neg_mask[n - 1:n, :]
    ps = jnp.sum(jnp.where(pos_last, sim_last, zero), axis=1, keepdims=True)
    pc = jnp.sum(pos_last.astype(jnp.float32), axis=1, keepdims=True)
    ns = jnp.sum(jnp.where(neg_last, sim_last, zero), axis=1, keepdims=True)
    nc = jnp.sum(neg_last.astype(jnp.float32), axis=1, keepdims=True)
    mp_ref[...] = ps / jnp.maximum(pc, 1.0)
    mn_ref[...] = ns / jnp.maximum(nc, 1.0)


def kernel(inputs, targets):
    n = inputs.shape[0]
    tcol = targets.reshape(n, 1)
    trow = targets.reshape(1, n)
    out_shape = [jax.ShapeDtypeStruct((1, 1), jnp.float32)] * 3
    loss, mp, mn = pl.pallas_call(
        _nca_kernel,
        out_shape=out_shape,
    )(inputs, tcol, trow)
    return loss[0, 0], jnp.float32(0.0), mp[0, 0], mn[0, 0]
